# baseline (speedup 1.0000x reference)
import functools

import jax
import jax.numpy as jnp
from jax import lax
from jax.experimental import pallas as pl
from jax.experimental.pallas import tpu as pltpu
from jax.experimental.pallas import tpu_sc as plsc

_NC = 2
_NS = 16
_NW = _NC * _NS


def _tc_copy_kernel(x_ref, a_ref):
    a_ref[...] = x_ref[...]


def _tc_copy(x):
    B, S, D = x.shape
    BB = 4
    blk = (BB, S, D)
    spec = pl.BlockSpec(blk, lambda i: (i, 0, 0))
    return pl.pallas_call(
        _tc_copy_kernel,
        grid=(B // BB,),
        in_specs=[spec],
        out_specs=spec,
        out_shape=jax.ShapeDtypeStruct(x.shape, x.dtype),
        compiler_params=pltpu.CompilerParams(
            dimension_semantics=("parallel",),
        ),
    )(x)


def _sc_copy(xf):
    R, D = xf.shape
    rows_per_w = R // _NW
    n_chunks = 8
    rows_per_chunk = rows_per_w // n_chunks

    mesh = plsc.VectorSubcoreMesh(core_axis_name="c", subcore_axis_name="s")

    @functools.partial(
        pl.kernel,
        mesh=mesh,
        out_type=jax.ShapeDtypeStruct((R, D), xf.dtype),
        scratch_types=[
            pltpu.VMEM((2, rows_per_chunk, D), xf.dtype),
            pltpu.SemaphoreType.DMA,
            pltpu.SemaphoreType.DMA,
        ],
    )
    def sc_copy_b(x_hbm, b_hbm, buf, in_sem, out_sem):
        wid = lax.axis_index("s") * _NC + lax.axis_index("c")
        base = wid * rows_per_w

        def sl(c):
            return pl.ds(base + c * rows_per_chunk, rows_per_chunk)

        ins = [
            pltpu.make_async_copy(x_hbm.at[sl(c)], buf.at[c % 2], in_sem)
            for c in range(n_chunks)
        ]
        outs = [
            pltpu.make_async_copy(buf.at[c % 2], b_hbm.at[sl(c)], out_sem)
            for c in range(n_chunks)
        ]
        ins[0].start()
        for c in range(n_chunks):
            if c + 1 < n_chunks:
                if c >= 1:
                    outs[c - 1].wait()
                ins[c + 1].start()
            ins[c].wait()
            outs[c].start()
        outs[n_chunks - 2].wait()
        outs[n_chunks - 1].wait()

    return sc_copy_b(xf)


def kernel(x, mask):
    B, S, D = x.shape
    a = _tc_copy(x)
    b = _sc_copy(x.reshape(B * S, D))
    return (a, b.reshape(B, S, D))


# speedup vs baseline: 2.0045x; 2.0045x over previous
import jax
import jax.numpy as jnp
from jax.experimental import pallas as pl
from jax.experimental.pallas import tpu as pltpu

_NCHUNK = 4


def _dma_staged_kernel(x_ref, a_ref, b_ref, bufs, in_sems, a_sems, b_sems):
    B = x_ref.shape[0]
    c = B // _NCHUNK
    ins, outs = [], []
    for i in range(_NCHUNK):
        sl = pl.ds(i * c, c)
        cp = pltpu.make_async_copy(x_ref.at[sl], bufs.at[i], in_sems.at[i])
        cp.start()
        ins.append(cp)
        outs.append((
            pltpu.make_async_copy(bufs.at[i], a_ref.at[sl], a_sems.at[i]),
            pltpu.make_async_copy(bufs.at[i], b_ref.at[sl], b_sems.at[i]),
        ))
    for i in range(_NCHUNK):
        ins[i].wait()
        outs[i][0].start()
        outs[i][1].start()
    for ca, cb in outs:
        ca.wait()
        cb.wait()


def kernel(x, mask):
    B, S, D = x.shape
    out = pl.pallas_call(
        _dma_staged_kernel,
        in_specs=[pl.BlockSpec(memory_space=pl.ANY)],
        out_specs=[
            pl.BlockSpec(memory_space=pl.ANY),
            pl.BlockSpec(memory_space=pl.ANY),
        ],
        out_shape=[
            jax.ShapeDtypeStruct(x.shape, x.dtype),
            jax.ShapeDtypeStruct(x.shape, x.dtype),
        ],
        scratch_shapes=[
            pltpu.VMEM((_NCHUNK, B // _NCHUNK, S, D), x.dtype),
            pltpu.SemaphoreType.DMA((_NCHUNK,)),
            pltpu.SemaphoreType.DMA((_NCHUNK,)),
            pltpu.SemaphoreType.DMA((_NCHUNK,)),
        ],
        compiler_params=pltpu.CompilerParams(
            vmem_limit_bytes=128 * 1024 * 1024,
        ),
    )(x)
    return (out[0], out[1])
